# baseline (speedup 1.0000x reference)
import jax
import jax.numpy as jnp
from jax.experimental import pallas as pl

S, B, D = 26, 4096, 128
W = D + 1
SG = 16
FB = SG * W
BN = 1024
GK = -(-S // SG)
GJ = B // BN


def _body(x_ref, t_ref, z_ref):
    for si in range(SG):
        xt = jnp.transpose(x_ref[si], (1, 0))
        z_ref[pl.ds(si * W, D), :] = xt
        tv = (1 - 2 * t_ref[si]).astype(jnp.float32)
        z_ref[pl.ds(si * W + D, 1), :] = tv[None, :]


_call = pl.pallas_call(
    _body,
    grid=(GK, GJ),
    in_specs=[
        pl.BlockSpec((SG, BN, D), lambda k, j: (k, j, 0)),
        pl.BlockSpec((SG, BN), lambda k, j: (k, j)),
    ],
    out_specs=pl.BlockSpec((FB, BN), lambda k, j: (k, j)),
    out_shape=jax.ShapeDtypeStruct((S * W, B), jnp.float32),
)


def kernel(x, tasks):
    return _call(x, tasks).T


# speedup vs baseline: 1.0016x; 1.0016x over previous
import jax
import jax.numpy as jnp
from jax.experimental import pallas as pl

S, B, D = 26, 4096, 128
W = D + 1
SG = 8
FB = SG * W
BN = 2048
GK = -(-S // SG)
GJ = B // BN


def _body(x_ref, t_ref, z_ref):
    for si in range(SG):
        xt = jnp.transpose(x_ref[si], (1, 0))
        z_ref[pl.ds(si * W, D), :] = xt
        tv = (1 - 2 * t_ref[si]).astype(jnp.float32)
        z_ref[pl.ds(si * W + D, 1), :] = tv[None, :]


_call = pl.pallas_call(
    _body,
    grid=(GK, GJ),
    in_specs=[
        pl.BlockSpec((SG, BN, D), lambda k, j: (k, j, 0)),
        pl.BlockSpec((SG, BN), lambda k, j: (k, j)),
    ],
    out_specs=pl.BlockSpec((FB, BN), lambda k, j: (k, j)),
    out_shape=jax.ShapeDtypeStruct((S * W, B), jnp.float32),
)


def kernel(x, tasks):
    return _call(x, tasks).T
